# baseline (speedup 1.0000x reference)
import functools

import jax
import jax.numpy as jnp
from jax import lax
from jax.experimental import pallas as pl
from jax.experimental.pallas import tpu as pltpu
from jax.experimental.pallas import tpu_sc as plsc

B = 32
T = 2048
D = 128
C = 256
NC = 2
NS = 16
NW = NC * NS
NCHUNK = T // C
LANES = 16


def _emb_body(x_hbm, tok_hbm, pos_hbm, out_hbm, idx_v, tok_v, pos_v, sem):
    wid = lax.axis_index("s") * NC + lax.axis_index("c")

    def chunk_body(ci, carry):
        base = wid * T + ci * C
        pltpu.sync_copy(x_hbm.at[pl.ds(base, C)], idx_v)
        pltpu.sync_copy(pos_hbm.at[pl.ds(ci * C, C)], pos_v)
        pltpu.async_copy(tok_hbm.at[idx_v], tok_v, sem).wait()

        def row_body(r, c2):
            for j in range(D // LANES):
                sl = pl.ds(j * LANES, LANES)
                tok_v[r, sl] = tok_v[r, sl] + pos_v[r, sl]
            return c2

        lax.fori_loop(0, C, row_body, 0)
        pltpu.sync_copy(tok_v, out_hbm.at[pl.ds(base, C)])
        return carry

    lax.fori_loop(0, NCHUNK, chunk_body, 0)


@jax.jit
def _emb_call(x_flat, token_table, pos_table):
    mesh = plsc.VectorSubcoreMesh(
        core_axis_name="c", subcore_axis_name="s", num_cores=NC, num_subcores=NS
    )
    f = pl.kernel(
        _emb_body,
        out_type=jax.ShapeDtypeStruct((B * T, D), jnp.float32),
        mesh=mesh,
        scratch_types=[
            pltpu.VMEM((C,), jnp.int32),
            pltpu.VMEM((C, D), jnp.float32),
            pltpu.VMEM((C, D), jnp.float32),
            pltpu.SemaphoreType.DMA,
        ],
    )
    return f(x_flat, token_table, pos_table)


def kernel(x, token_table, pos_table):
    x_flat = x.reshape(B * T).astype(jnp.int32)
    out = _emb_call(x_flat, token_table, pos_table)
    return out.reshape(B, T, D)


# speedup vs baseline: 1.8715x; 1.8715x over previous
import jax
import jax.numpy as jnp
from jax import lax
from jax.experimental import pallas as pl
from jax.experimental.pallas import tpu as pltpu
from jax.experimental.pallas import tpu_sc as plsc

B = 32
T = 2048
D = 128
C = 256
NC = 2
NS = 16
NW = NC * NS
NTC = T // C
NBG = NW // NTC
GB = B // NBG
LANES = 16


def _emb_body(x_hbm, tok_hbm, pos_hbm, out_hbm,
              pos_v, tok0, tok1, idx0, idx1,
              gsem0, gsem1, osem0, osem1):
    wid = lax.axis_index("s") * NC + lax.axis_index("c")
    tc = wid // NBG
    bg = wid % NBG

    toks = (tok0, tok1)
    idxs = (idx0, idx1)
    gsems = (gsem0, gsem1)
    osems = (osem0, osem1)

    def row_base(g):
        return (bg * GB + g) * T + tc * C

    pltpu.sync_copy(pos_hbm.at[pl.ds(tc * C, C)], pos_v)
    pltpu.sync_copy(x_hbm.at[pl.ds(row_base(0), C)], idxs[0])
    gathers = [pltpu.async_copy(tok_hbm.at[idxs[0]], toks[0], gsems[0])]
    out_copies = [None, None]

    def add_pos(tok_ref):
        def row_body(r, c2):
            for j in range(D // LANES):
                sl = pl.ds(j * LANES, LANES)
                plsc.addupdate(tok_ref.at[r, sl], pos_v[r, sl])
            return c2
        lax.fori_loop(0, C, row_body, 0, unroll=4)

    for g in range(GB):
        cur = g % 2
        nxt = 1 - cur
        if g + 1 < GB:
            pltpu.sync_copy(x_hbm.at[pl.ds(row_base(g + 1), C)], idxs[nxt])
            if out_copies[nxt] is not None:
                out_copies[nxt].wait()
            gathers.append(
                pltpu.async_copy(tok_hbm.at[idxs[nxt]], toks[nxt], gsems[nxt]))
        gathers[g].wait()
        add_pos(toks[cur])
        out_copies[cur] = pltpu.async_copy(
            toks[cur], out_hbm.at[pl.ds(row_base(g), C)], osems[cur])

    out_copies[0].wait()
    out_copies[1].wait()


@jax.jit
def _emb_call(x_flat, token_table, pos_table):
    mesh = plsc.VectorSubcoreMesh(
        core_axis_name="c", subcore_axis_name="s", num_cores=NC, num_subcores=NS
    )
    f = pl.kernel(
        _emb_body,
        out_type=jax.ShapeDtypeStruct((B * T, D), jnp.float32),
        mesh=mesh,
        scratch_types=[
            pltpu.VMEM((C, D), jnp.float32),
            pltpu.VMEM((C, D), jnp.float32),
            pltpu.VMEM((C, D), jnp.float32),
            pltpu.VMEM((C,), jnp.int32),
            pltpu.VMEM((C,), jnp.int32),
            pltpu.SemaphoreType.DMA,
            pltpu.SemaphoreType.DMA,
            pltpu.SemaphoreType.DMA,
            pltpu.SemaphoreType.DMA,
        ],
    )
    return f(x_flat, token_table, pos_table)


def kernel(x, token_table, pos_table):
    x_flat = x.reshape(B * T).astype(jnp.int32)
    out = _emb_call(x_flat, token_table, pos_table)
    return out.reshape(B, T, D)


# speedup vs baseline: 2.0327x; 1.0861x over previous
import jax
import jax.numpy as jnp
from jax import lax
from jax.experimental import pallas as pl
from jax.experimental.pallas import tpu as pltpu
from jax.experimental.pallas import tpu_sc as plsc

B = 32
T = 2048
D = 128
C = 128
NC = 2
NS = 16
NW = NC * NS
NTC = T // C
NBG = NW // NTC
GB = B // NBG
LANES = 16
NBUF = 4
DEPTH = 3


def _emb_body(x_hbm, tok_hbm, pos_hbm, out_hbm,
              pos_v, idx_v, tok0, tok1, tok2, tok3,
              gsem0, gsem1, gsem2, gsem3,
              osem0, osem1, osem2, osem3):
    wid = lax.axis_index("s") * NC + lax.axis_index("c")
    tc = wid // NBG
    bg = wid % NBG

    toks = (tok0, tok1, tok2, tok3)
    gsems = (gsem0, gsem1, gsem2, gsem3)
    osems = (osem0, osem1, osem2, osem3)

    def row_base(g):
        return (bg * GB + g) * T + tc * C

    pltpu.sync_copy(
        x_hbm.at[pl.ds(bg * GB, GB), pl.ds(tc * C, C)], idx_v)
    pltpu.sync_copy(pos_hbm.at[pl.ds(tc * C, C)], pos_v)

    def add_pos(tok_ref):
        def row_body(r, c2):
            for j in range(D // LANES):
                sl = pl.ds(j * LANES, LANES)
                plsc.addupdate(tok_ref.at[r, sl], pos_v[r, sl])
            return c2
        lax.fori_loop(0, C, row_body, 0, unroll=4)

    gathers = [None] * GB
    out_copies = [None] * NBUF

    def issue_gather(g):
        buf = g % NBUF
        if out_copies[buf] is not None:
            out_copies[buf].wait()
            out_copies[buf] = None
        gathers[g] = pltpu.async_copy(
            tok_hbm.at[idx_v.at[g]], toks[buf], gsems[buf])

    for p in range(DEPTH):
        issue_gather(p)
    for g in range(GB):
        buf = g % NBUF
        gathers[g].wait()
        add_pos(toks[buf])
        out_copies[buf] = pltpu.async_copy(
            toks[buf], out_hbm.at[pl.ds(row_base(g), C)], osems[buf])
        if g + DEPTH < GB:
            issue_gather(g + DEPTH)

    for oc in out_copies:
        if oc is not None:
            oc.wait()


@jax.jit
def _emb_call(x2d, token_table, pos_table):
    mesh = plsc.VectorSubcoreMesh(
        core_axis_name="c", subcore_axis_name="s", num_cores=NC, num_subcores=NS
    )
    f = pl.kernel(
        _emb_body,
        out_type=jax.ShapeDtypeStruct((B * T, D), jnp.float32),
        mesh=mesh,
        scratch_types=[
            pltpu.VMEM((C, D), jnp.float32),
            pltpu.VMEM((GB, C), jnp.int32),
            pltpu.VMEM((C, D), jnp.float32),
            pltpu.VMEM((C, D), jnp.float32),
            pltpu.VMEM((C, D), jnp.float32),
            pltpu.VMEM((C, D), jnp.float32),
            pltpu.SemaphoreType.DMA,
            pltpu.SemaphoreType.DMA,
            pltpu.SemaphoreType.DMA,
            pltpu.SemaphoreType.DMA,
            pltpu.SemaphoreType.DMA,
            pltpu.SemaphoreType.DMA,
            pltpu.SemaphoreType.DMA,
            pltpu.SemaphoreType.DMA,
        ],
    )
    return f(x2d, token_table, pos_table)


def kernel(x, token_table, pos_table):
    out = _emb_call(x.astype(jnp.int32), token_table, pos_table)
    return out.reshape(B, T, D)
